# baseline (speedup 1.0000x reference)
import jax
import jax.numpy as jnp
from jax.experimental import pallas as pl

EPS = 1e-8
NUM_CLASSES = 100
K_STATIC = 2048


def kernel(inputs, labels, features, k):
    N = features.shape[0]
    counts = jnp.bincount(labels, length=NUM_CLASSES).astype(features.dtype)
    class_sum = jax.ops.segment_sum(features, labels, num_segments=NUM_CLASSES)
    centroids = class_sum / jnp.maximum(counts, 1.0)[:, None]
    cent_i = centroids[labels]
    f_norm = jnp.linalg.norm(features, axis=1)
    c_norm = jnp.linalg.norm(cent_i, axis=1)
    dist_to_centroid = jnp.sum(features * cent_i, axis=1) / (
        jnp.maximum(f_norm, EPS) * jnp.maximum(c_norm, EPS)
    )
    fn = features / jnp.maximum(f_norm, EPS)[:, None]
    S = fn @ fn.T
    div_mean = (jnp.sum(S, axis=1) - jnp.diag(S)) / (N - 1)
    diversity = 1.0 - div_mean
    quality = dist_to_centroid * 0.7 + diversity * 0.3
    quality = quality + 0.0 * jnp.asarray(k, dtype=quality.dtype)
    _, idx = jax.lax.top_k(quality, K_STATIC)
    return (inputs[idx], labels[idx])


# speedup vs baseline: 1.3823x; 1.3823x over previous
import functools

import jax
import jax.numpy as jnp
from jax import lax
from jax.experimental import pallas as pl
from jax.experimental.pallas import tpu as pltpu
from jax.experimental.pallas import tpu_sc as plsc

EPS = 1e-8
NUM_CLASSES = 100
K_STATIC = 2048
N = 8192
D = 64
HI = lax.Precision.HIGHEST


def _rowdot(x):
    acc = x[0:8] + x[8:16]
    for k in range(2, 8):
        acc = acc + x[8 * k:8 * k + 8]
    t = acc[0:4] + acc[4:8]
    t = t[0:2] + t[2:4]
    t = t[0:1] + t[1:2]
    return t


def _tc_kernel(ft_ref, labr_ref, centt_ref, out_ref):
    ft = ft_ref[...]
    labr = labr_ref[...]
    centt = centt_ref[...]

    onehot = (lax.broadcasted_iota(jnp.int32, (128, N), 0) == labr).astype(jnp.float32)
    ct = lax.dot_general(centt, onehot, (((1,), (0,)), ((), ())), precision=HI)

    nsqf = _rowdot(ft * ft)
    nsqc = _rowdot(ct * ct)
    prod = _rowdot(ft * ct)
    f_norm = jnp.sqrt(nsqf)
    c_norm = jnp.sqrt(nsqc)
    maxf = jnp.maximum(f_norm, EPS)
    maxc = jnp.maximum(c_norm, EPS)
    dist = prod / (maxf * maxc)
    fnt = ft / maxf
    fnb = fnt.astype(jnp.bfloat16).astype(jnp.float32)
    g = jnp.sum(fnb, axis=1, keepdims=True)
    row_sum = _rowdot(fnb * g)
    diag = _rowdot(fnb * fnb)
    div_mean = (row_sum - diag) / jnp.float32(N - 1)
    quality = dist * jnp.float32(0.7) + (jnp.float32(1.0) - div_mean) * jnp.float32(0.3)

    qi = lax.bitcast_convert_type(quality, jnp.int32)
    v = jnp.where(qi >= 0, qi, jnp.int32(-1) - (qi ^ jnp.int32(-2147483648)))
    idx = lax.broadcasted_iota(jnp.int32, (1, N), 1)

    def tbody(b, t):
        trial = jnp.where(b == 0, jnp.int32(0), t + (jnp.int32(1) << (31 - b)))
        cnt = jnp.sum((v >= trial).astype(jnp.int32))
        return jnp.where(cnt >= K_STATIC, trial, t)

    t = lax.fori_loop(0, 32, tbody, jnp.int32(-2147483648))
    count_gt = jnp.sum((v > t).astype(jnp.int32))
    need = K_STATIC - count_gt
    ties = (v == t)

    def cbody(b, c):
        trial = c + (jnp.int32(1) << (12 - b))
        cnt = jnp.sum((ties & (idx <= trial)).astype(jnp.int32))
        return jnp.where(cnt < need, trial, c)

    c = lax.fori_loop(0, 13, cbody, jnp.int32(-1))
    sel = (v > t) | (ties & (idx <= c + 1))
    self32 = sel.astype(jnp.float32)

    pinc = self32
    for s in [1, 2, 4, 8, 16, 32, 64, 128, 256, 512, 1024, 2048, 4096]:
        pinc = pinc + jnp.concatenate(
            [jnp.zeros((1, s), jnp.float32), pinc[:, :N - s]], axis=1)
    p = pinc - self32

    b3 = ((v >> 24) + 128).astype(jnp.float32)
    r24 = v & jnp.int32(0x00FFFFFF)
    b2 = (r24 >> 16).astype(jnp.float32)
    b1 = ((r24 >> 8) & 255).astype(jnp.float32)
    b0 = (r24 & 255).astype(jnp.float32)
    ih = (idx >> 8).astype(jnp.float32)
    il = (idx & 255).astype(jnp.float32)
    payload = jnp.concatenate([b3, b2, b1, b0, ih, il, jnp.zeros((2, N), jnp.float32)], axis=0)

    j_iota = lax.broadcasted_iota(jnp.int32, (K_STATIC, N), 0)
    m_t = jnp.where((p.astype(jnp.int32) == j_iota) & sel, 1.0, 0.0)
    comp = lax.dot_general(m_t, payload, (((1,), (1,)), ((), ())))
    cb3 = comp[:, 0:1]
    cb2 = comp[:, 1:2]
    cb1 = comp[:, 2:3]
    cb0 = comp[:, 3:4]
    cih = comp[:, 4:5]
    cil = comp[:, 5:6]
    cvh = cb3 * 256.0 + cb2
    cvl = cb1 * 256.0 + cb0
    cidx = cih * 256.0 + cil

    rvh = jnp.transpose(cvh)
    rvl = jnp.transpose(cvl)
    ridx = jnp.transpose(cidx)
    gt = (rvh > cvh) | ((rvh == cvh) & ((rvl > cvl) | ((rvl == cvl) & (ridx < cidx))))
    rank = jnp.sum(gt.astype(jnp.float32), axis=1, keepdims=True)

    p_t = jnp.where(jnp.transpose(rank).astype(jnp.int32)
                    == lax.broadcasted_iota(jnp.int32, (K_STATIC, K_STATIC), 0), 1.0, 0.0)
    outcols = jnp.concatenate([cih, cil, jnp.zeros((K_STATIC, 6), jnp.float32)], axis=1)
    res = lax.dot_general(p_t, outcols, (((1,), (0,)), ((), ())))
    out_ref[...] = res


def _tc_topk(features_t, labels_row, cent_t_pad):
    return pl.pallas_call(
        _tc_kernel,
        out_shape=jax.ShapeDtypeStruct((K_STATIC, 8), jnp.float32),
    )(features_t, labels_row, cent_t_pad)


def _sc_gather(table, idx):
    info = plsc.get_sparse_core_info()
    nw = info.num_cores * info.num_subcores
    b_per_w = K_STATIC // nw
    mesh = plsc.VectorSubcoreMesh(core_axis_name="c", subcore_axis_name="s")

    @functools.partial(
        pl.kernel, mesh=mesh,
        out_type=jax.ShapeDtypeStruct((K_STATIC, 128), jnp.float32),
        scratch_types=[
            pltpu.VMEM((b_per_w,), jnp.int32),
            pltpu.VMEM((b_per_w, 128), jnp.float32),
            pltpu.SemaphoreType.DMA,
        ],
    )
    def k(table_hbm, idx_hbm, out_hbm, idx_v, rows_v, sem):
        wid = lax.axis_index("s") * info.num_cores + lax.axis_index("c")
        base = wid * b_per_w
        pltpu.sync_copy(idx_hbm.at[pl.ds(base, b_per_w)], idx_v)
        pltpu.async_copy(table_hbm.at[idx_v], rows_v, sem).wait()
        pltpu.sync_copy(rows_v, out_hbm.at[pl.ds(base, b_per_w)])

    return k(table, idx)


def kernel(inputs, labels, features, k):
    counts = jnp.bincount(labels, length=NUM_CLASSES).astype(features.dtype)
    class_sum = jax.ops.segment_sum(features, labels, num_segments=NUM_CLASSES)
    centroids = class_sum / jnp.maximum(counts, 1.0)[:, None]
    cent_pad = jnp.pad(centroids, ((0, 128 - NUM_CLASSES), (0, 0)))
    res = _tc_topk(features.T, labels[None, :].astype(jnp.int32), cent_pad.T)
    out_idx = (res[:, 0] * 256.0 + res[:, 1]).astype(jnp.int32)
    table = jnp.pad(inputs, ((0, 0), (0, 128 - D)))
    rows = _sc_gather(table, out_idx)[:, :D]
    out_labels = labels[out_idx]
    return (rows, out_labels)


# speedup vs baseline: 1.4767x; 1.0683x over previous
import functools

import jax
import jax.numpy as jnp
from jax import lax
from jax.experimental import pallas as pl
from jax.experimental.pallas import tpu as pltpu
from jax.experimental.pallas import tpu_sc as plsc

EPS = 1e-8
NUM_CLASSES = 100
K_STATIC = 2048
N = 8192
D = 64
HI = lax.Precision.HIGHEST


def _rowdot(x):
    acc = x[0:8] + x[8:16]
    for k in range(2, 8):
        acc = acc + x[8 * k:8 * k + 8]
    t = acc[0:4] + acc[4:8]
    t = t[0:2] + t[2:4]
    t = t[0:1] + t[1:2]
    return t


def _tc_kernel(ft_ref, labr_ref, cst_ref, out_ref):
    ft = ft_ref[...]
    labr = labr_ref[...]
    cst = cst_ref[...]

    onehot = (lax.broadcasted_iota(jnp.int32, (128, N), 0) == labr).astype(jnp.float32)
    counts = jnp.transpose(jnp.sum(onehot, axis=1, keepdims=True))
    centt = cst / jnp.maximum(counts, 1.0)
    ct = lax.dot_general(centt, onehot, (((1,), (0,)), ((), ())), precision=HI)

    nsqf = _rowdot(ft * ft)
    nsqc = _rowdot(ct * ct)
    prod = _rowdot(ft * ct)
    f_norm = jnp.sqrt(nsqf)
    c_norm = jnp.sqrt(nsqc)
    maxf = jnp.maximum(f_norm, EPS)
    maxc = jnp.maximum(c_norm, EPS)
    dist = prod / (maxf * maxc)
    fnt = ft / maxf
    fnb = fnt.astype(jnp.bfloat16).astype(jnp.float32)
    g = jnp.sum(fnb, axis=1, keepdims=True)
    row_sum = _rowdot(fnb * g)
    diag = _rowdot(fnb * fnb)
    div_mean = (row_sum - diag) / jnp.float32(N - 1)
    quality = dist * jnp.float32(0.7) + (jnp.float32(1.0) - div_mean) * jnp.float32(0.3)

    qi = lax.bitcast_convert_type(quality, jnp.int32)
    v = jnp.where(qi >= 0, qi, jnp.int32(-1) - (qi ^ jnp.int32(-2147483648)))
    idx = lax.broadcasted_iota(jnp.int32, (1, N), 1)

    def tbody(b, t):
        trial = jnp.where(b == 0, jnp.int32(0), t + (jnp.int32(1) << (31 - b)))
        cnt = jnp.sum((v >= trial).astype(jnp.int32))
        return jnp.where(cnt >= K_STATIC, trial, t)

    t = lax.fori_loop(0, 32, tbody, jnp.int32(-2147483648))
    count_gt = jnp.sum((v > t).astype(jnp.int32))
    need = K_STATIC - count_gt
    ties = (v == t)

    def cbody(b, c):
        trial = c + (jnp.int32(1) << (12 - b))
        cnt = jnp.sum((ties & (idx <= trial)).astype(jnp.int32))
        return jnp.where(cnt < need, trial, c)

    c = lax.fori_loop(0, 13, cbody, jnp.int32(-1))
    sel = (v > t) | (ties & (idx <= c + 1))
    self32 = sel.astype(jnp.float32)

    pinc = self32
    for s in [1, 2, 4, 8, 16, 32, 64, 128, 256, 512, 1024, 2048, 4096]:
        pinc = pinc + jnp.concatenate(
            [jnp.zeros((1, s), jnp.float32), pinc[:, :N - s]], axis=1)
    p = pinc - self32

    b3 = ((v >> 24) + 128).astype(jnp.float32)
    r24 = v & jnp.int32(0x00FFFFFF)
    b2 = (r24 >> 16).astype(jnp.float32)
    b1 = ((r24 >> 8) & 255).astype(jnp.float32)
    b0 = (r24 & 255).astype(jnp.float32)
    ih = (idx >> 8).astype(jnp.float32)
    il = (idx & 255).astype(jnp.float32)
    payload = jnp.concatenate([b3, b2, b1, b0, ih, il, jnp.zeros((2, N), jnp.float32)], axis=0)

    j_iota = lax.broadcasted_iota(jnp.int32, (K_STATIC, N), 0)
    m_t = jnp.where((p.astype(jnp.int32) == j_iota) & sel, 1.0, 0.0)
    comp = lax.dot_general(m_t, payload, (((1,), (1,)), ((), ())))
    cb3 = comp[:, 0:1]
    cb2 = comp[:, 1:2]
    cb1 = comp[:, 2:3]
    cb0 = comp[:, 3:4]
    cih = comp[:, 4:5]
    cil = comp[:, 5:6]
    cvh = cb3 * 256.0 + cb2
    cvl = cb1 * 256.0 + cb0
    cidx = cih * 256.0 + cil

    rvh = jnp.transpose(cvh)
    rvl = jnp.transpose(cvl)
    ridx = jnp.transpose(cidx)
    gt = (rvh > cvh) | ((rvh == cvh) & ((rvl > cvl) | ((rvl == cvl) & (ridx < cidx))))
    rank = jnp.sum(gt.astype(jnp.float32), axis=1, keepdims=True)

    p_t = jnp.where(jnp.transpose(rank).astype(jnp.int32)
                    == lax.broadcasted_iota(jnp.int32, (K_STATIC, K_STATIC), 0), 1.0, 0.0)
    outcols = jnp.concatenate([cih, cil, jnp.zeros((K_STATIC, 6), jnp.float32)], axis=1)
    res = lax.dot_general(p_t, outcols, (((1,), (0,)), ((), ())))
    out_ref[...] = res


def _tc_topk(features_t, labels_row, cent_t_pad):
    return pl.pallas_call(
        _tc_kernel,
        out_shape=jax.ShapeDtypeStruct((K_STATIC, 8), jnp.float32),
    )(features_t, labels_row, cent_t_pad)


def _sc_gather(table, idx):
    info = plsc.get_sparse_core_info()
    nw = info.num_cores * info.num_subcores
    b_per_w = K_STATIC // nw
    mesh = plsc.VectorSubcoreMesh(core_axis_name="c", subcore_axis_name="s")

    @functools.partial(
        pl.kernel, mesh=mesh,
        out_type=jax.ShapeDtypeStruct((K_STATIC, 128), jnp.float32),
        scratch_types=[
            pltpu.VMEM((b_per_w,), jnp.int32),
            pltpu.VMEM((b_per_w, 128), jnp.float32),
            pltpu.SemaphoreType.DMA,
        ],
    )
    def k(table_hbm, idx_hbm, out_hbm, idx_v, rows_v, sem):
        wid = lax.axis_index("s") * info.num_cores + lax.axis_index("c")
        base = wid * b_per_w
        pltpu.sync_copy(idx_hbm.at[pl.ds(base, b_per_w)], idx_v)
        pltpu.async_copy(table_hbm.at[idx_v], rows_v, sem).wait()
        pltpu.sync_copy(rows_v, out_hbm.at[pl.ds(base, b_per_w)])

    return k(table, idx)


def kernel(inputs, labels, features, k):
    class_sum = jax.ops.segment_sum(features, labels, num_segments=NUM_CLASSES)
    cs_pad = jnp.pad(class_sum, ((0, 128 - NUM_CLASSES), (0, 0)))
    res = _tc_topk(features.T, labels[None, :].astype(jnp.int32), cs_pad.T)
    out_idx = (res[:, 0] * 256.0 + res[:, 1]).astype(jnp.int32)
    table = jnp.pad(inputs, ((0, 0), (0, 128 - D)))
    rows = _sc_gather(table, out_idx)[:, :D]
    out_labels = labels[out_idx]
    return (rows, out_labels)


# speedup vs baseline: 1.4994x; 1.0154x over previous
import functools

import jax
import jax.numpy as jnp
from jax import lax
from jax.experimental import pallas as pl
from jax.experimental.pallas import tpu as pltpu
from jax.experimental.pallas import tpu_sc as plsc

EPS = 1e-8
NUM_CLASSES = 100
K_STATIC = 2048
N = 8192
D = 64
HI = lax.Precision.HIGHEST


def _rowdot(x):
    acc = x[0:8] + x[8:16]
    for k in range(2, 8):
        acc = acc + x[8 * k:8 * k + 8]
    t = acc[0:4] + acc[4:8]
    t = t[0:2] + t[2:4]
    t = t[0:1] + t[1:2]
    return t


def _tc_diversity_kernel(ft_ref, out_ref):
    ft = ft_ref[...]
    nsqf = _rowdot(ft * ft)
    f_norm = jnp.sqrt(nsqf)
    maxf = jnp.maximum(f_norm, EPS)
    fnt = ft / maxf
    fnb = fnt.astype(jnp.bfloat16).astype(jnp.float32)
    g = jnp.sum(fnb, axis=1, keepdims=True)
    row_sum = _rowdot(fnb * g)
    diag = _rowdot(fnb * fnb)
    div_mean = (row_sum - diag) / jnp.float32(N - 1)
    part_b = (jnp.float32(1.0) - div_mean) * jnp.float32(0.3)
    out_ref[...] = jnp.concatenate(
        [maxf, part_b, jnp.zeros((6, N), jnp.float32)], axis=0)


def _tc_kernel(ft_ref, labr_ref, cst_ref, div_ref, out_ref):
    ft = ft_ref[...]
    labr = labr_ref[...]
    cst = cst_ref[...]
    maxf = div_ref[0:1, :]
    part_b = div_ref[1:2, :]

    onehot = (lax.broadcasted_iota(jnp.int32, (128, N), 0) == labr).astype(jnp.float32)
    counts = jnp.transpose(jnp.sum(onehot, axis=1, keepdims=True))
    centt = cst / jnp.maximum(counts, 1.0)
    ct = lax.dot_general(centt, onehot, (((1,), (0,)), ((), ())), precision=HI)

    nsqc = _rowdot(ct * ct)
    prod = _rowdot(ft * ct)
    c_norm = jnp.sqrt(nsqc)
    maxc = jnp.maximum(c_norm, EPS)
    dist = prod / (maxf * maxc)
    quality = dist * jnp.float32(0.7) + part_b

    qi = lax.bitcast_convert_type(quality, jnp.int32)
    v = jnp.where(qi >= 0, qi, jnp.int32(-1) - (qi ^ jnp.int32(-2147483648)))
    idx = lax.broadcasted_iota(jnp.int32, (1, N), 1)

    def tbody(b, t):
        trial = jnp.where(b == 0, jnp.int32(0), t + (jnp.int32(1) << (31 - b)))
        cnt = jnp.sum((v >= trial).astype(jnp.int32))
        return jnp.where(cnt >= K_STATIC, trial, t)

    t = lax.fori_loop(0, 32, tbody, jnp.int32(-2147483648))
    count_gt = jnp.sum((v > t).astype(jnp.int32))
    need = K_STATIC - count_gt
    ties = (v == t)

    def cbody(b, c):
        trial = c + (jnp.int32(1) << (12 - b))
        cnt = jnp.sum((ties & (idx <= trial)).astype(jnp.int32))
        return jnp.where(cnt < need, trial, c)

    c = lax.fori_loop(0, 13, cbody, jnp.int32(-1))
    sel = (v > t) | (ties & (idx <= c + 1))
    self32 = sel.astype(jnp.float32)

    pinc = self32
    for s in [1, 2, 4, 8, 16, 32, 64, 128, 256, 512, 1024, 2048, 4096]:
        pinc = pinc + jnp.concatenate(
            [jnp.zeros((1, s), jnp.float32), pinc[:, :N - s]], axis=1)
    p = pinc - self32

    b3 = ((v >> 24) + 128).astype(jnp.float32)
    r24 = v & jnp.int32(0x00FFFFFF)
    b2 = (r24 >> 16).astype(jnp.float32)
    b1 = ((r24 >> 8) & 255).astype(jnp.float32)
    b0 = (r24 & 255).astype(jnp.float32)
    ih = (idx >> 8).astype(jnp.float32)
    il = (idx & 255).astype(jnp.float32)
    payload = jnp.concatenate([b3, b2, b1, b0, ih, il, jnp.zeros((2, N), jnp.float32)], axis=0)

    j_iota = lax.broadcasted_iota(jnp.int32, (K_STATIC, N), 0)
    m_t = jnp.where((p.astype(jnp.int32) == j_iota) & sel, 1.0, 0.0)
    comp = lax.dot_general(m_t, payload, (((1,), (1,)), ((), ())))
    cb3 = comp[:, 0:1]
    cb2 = comp[:, 1:2]
    cb1 = comp[:, 2:3]
    cb0 = comp[:, 3:4]
    cih = comp[:, 4:5]
    cil = comp[:, 5:6]
    cvh = cb3 * 256.0 + cb2
    cvl = cb1 * 256.0 + cb0
    cidx = cih * 256.0 + cil

    rvh = jnp.transpose(cvh)
    rvl = jnp.transpose(cvl)
    ridx = jnp.transpose(cidx)
    gt = (rvh > cvh) | ((rvh == cvh) & ((rvl > cvl) | ((rvl == cvl) & (ridx < cidx))))
    rank = jnp.sum(gt.astype(jnp.float32), axis=1, keepdims=True)

    p_t = jnp.where(jnp.transpose(rank).astype(jnp.int32)
                    == lax.broadcasted_iota(jnp.int32, (K_STATIC, K_STATIC), 0), 1.0, 0.0)
    outcols = jnp.concatenate([cih, cil, jnp.zeros((K_STATIC, 6), jnp.float32)], axis=1)
    res = lax.dot_general(p_t, outcols, (((1,), (0,)), ((), ())))
    out_ref[...] = res


def _tc_topk(features_t, labels_row, cent_t_pad):
    div = pl.pallas_call(
        _tc_diversity_kernel,
        out_shape=jax.ShapeDtypeStruct((8, N), jnp.float32),
    )(features_t)
    return pl.pallas_call(
        _tc_kernel,
        out_shape=jax.ShapeDtypeStruct((K_STATIC, 8), jnp.float32),
    )(features_t, labels_row, cent_t_pad, div)


def _sc_gather(table, idx):
    info = plsc.get_sparse_core_info()
    nw = info.num_cores * info.num_subcores
    b_per_w = K_STATIC // nw
    mesh = plsc.VectorSubcoreMesh(core_axis_name="c", subcore_axis_name="s")

    @functools.partial(
        pl.kernel, mesh=mesh,
        out_type=jax.ShapeDtypeStruct((K_STATIC, 128), jnp.float32),
        scratch_types=[
            pltpu.VMEM((b_per_w,), jnp.int32),
            pltpu.VMEM((b_per_w, 128), jnp.float32),
            pltpu.SemaphoreType.DMA,
        ],
    )
    def k(table_hbm, idx_hbm, out_hbm, idx_v, rows_v, sem):
        wid = lax.axis_index("s") * info.num_cores + lax.axis_index("c")
        base = wid * b_per_w
        pltpu.sync_copy(idx_hbm.at[pl.ds(base, b_per_w)], idx_v)
        pltpu.async_copy(table_hbm.at[idx_v], rows_v, sem).wait()
        pltpu.sync_copy(rows_v, out_hbm.at[pl.ds(base, b_per_w)])

    return k(table, idx)


def kernel(inputs, labels, features, k):
    class_sum = jax.ops.segment_sum(features, labels, num_segments=NUM_CLASSES)
    cs_pad = jnp.pad(class_sum, ((0, 128 - NUM_CLASSES), (0, 0)))
    res = _tc_topk(features.T, labels[None, :].astype(jnp.int32), cs_pad.T)
    out_idx = (res[:, 0] * 256.0 + res[:, 1]).astype(jnp.int32)
    table = jnp.pad(inputs, ((0, 0), (0, 128 - D)))
    rows = _sc_gather(table, out_idx)[:, :D]
    out_labels = labels[out_idx]
    return (rows, out_labels)
